# Initial kernel scaffold; baseline (speedup 1.0000x reference)
import jax
import jax.numpy as jnp
from jax.experimental import pallas as pl


def kernel(z, codebooks):
    raise NotImplementedError("write your pallas kernel here")


# speedup vs baseline: 2.8423x; 2.8423x over previous
import functools

import jax
import jax.numpy as jnp
from jax.experimental import pallas as pl

NUM_HEADS = 8
NUM_EMBEDDINGS = 512
EMBEDDING_DIM = 256
HEAD_DIM = EMBEDDING_DIM // NUM_HEADS


def _pq_kernel(z_ref, cb_ref, cbt_ref, zq_ref, idx_ref, dist_ref):
    z_blk = z_ref[...]
    idx_cols = []
    zq_cols = []
    for h in range(NUM_HEADS):
        zh = z_blk[:, h * HEAD_DIM:(h + 1) * HEAD_DIM]
        ch = cb_ref[h]
        cht = cbt_ref[h]
        z_sq = jnp.sum(zh * zh, axis=-1, keepdims=True)
        c_sq = jnp.sum(ch * ch, axis=-1)[None, :]
        dot = jnp.dot(zh, cht, preferred_element_type=jnp.float32)
        dist = z_sq + c_sq - 2.0 * dot
        dist_ref[:, h * NUM_EMBEDDINGS:(h + 1) * NUM_EMBEDDINGS] = dist
        idx = jnp.argmin(dist, axis=-1).astype(jnp.int32)
        idx_cols.append(idx[:, None])
        onehot = (jax.lax.broadcasted_iota(jnp.int32, dist.shape, 1)
                  == idx[:, None]).astype(jnp.float32)
        zq_h = jnp.dot(onehot, ch, preferred_element_type=jnp.float32)
        zq_cols.append(zh + (zq_h - zh))
    idx_ref[...] = jnp.concatenate(idx_cols, axis=1)
    zq_ref[...] = jnp.concatenate(zq_cols, axis=1)


@functools.partial(jax.jit, static_argnames=("block_b",))
def _pq(z, codebooks, block_b=512):
    bsz, dim = z.shape
    cbt = jnp.transpose(codebooks, (0, 2, 1))
    grid = (bsz // block_b,)
    zq, idx, dist = pl.pallas_call(
        _pq_kernel,
        grid=grid,
        in_specs=[
            pl.BlockSpec((block_b, dim), lambda i: (i, 0)),
            pl.BlockSpec((NUM_HEADS, NUM_EMBEDDINGS, HEAD_DIM), lambda i: (0, 0, 0)),
            pl.BlockSpec((NUM_HEADS, HEAD_DIM, NUM_EMBEDDINGS), lambda i: (0, 0, 0)),
        ],
        out_specs=[
            pl.BlockSpec((block_b, dim), lambda i: (i, 0)),
            pl.BlockSpec((block_b, NUM_HEADS), lambda i: (i, 0)),
            pl.BlockSpec((block_b, NUM_HEADS * NUM_EMBEDDINGS), lambda i: (i, 0)),
        ],
        out_shape=[
            jax.ShapeDtypeStruct((bsz, dim), jnp.float32),
            jax.ShapeDtypeStruct((bsz, NUM_HEADS), jnp.int32),
            jax.ShapeDtypeStruct((bsz, NUM_HEADS * NUM_EMBEDDINGS), jnp.float32),
        ],
    )(z, codebooks, cbt)
    return zq, idx, dist.reshape(bsz, NUM_HEADS, NUM_EMBEDDINGS)


def kernel(z, codebooks):
    return _pq(z, codebooks)


# speedup vs baseline: 3.4662x; 1.2195x over previous
import functools

import jax
import jax.numpy as jnp
from jax.experimental import pallas as pl

NUM_HEADS = 8
NUM_EMBEDDINGS = 512
EMBEDDING_DIM = 256
HEAD_DIM = EMBEDDING_DIM // NUM_HEADS


def _pq_kernel(z_ref, cb_ref, acb_ref, zq_ref, idx_ref, dist_ref):
    z_blk = z_ref[...]
    bb = z_blk.shape[0]
    ones = jnp.ones((bb, 1), dtype=jnp.float32)
    idx_cols = []
    zq_cols = []
    for h in range(NUM_HEADS):
        zh = z_blk[:, h * HEAD_DIM:(h + 1) * HEAD_DIM]
        ch = cb_ref[h]
        azh = jnp.concatenate([zh, zh * zh, ones], axis=1)
        dist = jnp.dot(azh, acb_ref[h],
                       preferred_element_type=jnp.float32)
        dist_ref[:, h * NUM_EMBEDDINGS:(h + 1) * NUM_EMBEDDINGS] = dist
        idx = jnp.argmin(dist, axis=-1).astype(jnp.int32)
        idx_cols.append(idx[:, None])
        onehot = (jax.lax.broadcasted_iota(jnp.int32, dist.shape, 1)
                  == idx[:, None]).astype(jnp.float32)
        zq_h = jnp.dot(onehot, ch, preferred_element_type=jnp.float32)
        zq_cols.append(zh + (zq_h - zh))
    idx_ref[...] = jnp.concatenate(idx_cols, axis=1)
    zq_ref[...] = jnp.concatenate(zq_cols, axis=1)


@functools.partial(jax.jit, static_argnames=("block_b",))
def _pq(z, codebooks, block_b=512):
    bsz, dim = z.shape
    cbt = jnp.transpose(codebooks, (0, 2, 1))
    c_sq = jnp.sum(codebooks * codebooks, axis=-1)
    acb = jnp.concatenate(
        [-2.0 * cbt,
         jnp.ones((NUM_HEADS, HEAD_DIM, NUM_EMBEDDINGS), jnp.float32),
         c_sq[:, None, :]], axis=1)
    grid = (bsz // block_b,)
    zq, idx, dist = pl.pallas_call(
        _pq_kernel,
        grid=grid,
        in_specs=[
            pl.BlockSpec((block_b, dim), lambda i: (i, 0)),
            pl.BlockSpec((NUM_HEADS, NUM_EMBEDDINGS, HEAD_DIM), lambda i: (0, 0, 0)),
            pl.BlockSpec((NUM_HEADS, 2 * HEAD_DIM + 1, NUM_EMBEDDINGS), lambda i: (0, 0, 0)),
        ],
        out_specs=[
            pl.BlockSpec((block_b, dim), lambda i: (i, 0)),
            pl.BlockSpec((block_b, NUM_HEADS), lambda i: (i, 0)),
            pl.BlockSpec((block_b, NUM_HEADS * NUM_EMBEDDINGS), lambda i: (i, 0)),
        ],
        out_shape=[
            jax.ShapeDtypeStruct((bsz, dim), jnp.float32),
            jax.ShapeDtypeStruct((bsz, NUM_HEADS), jnp.int32),
            jax.ShapeDtypeStruct((bsz, NUM_HEADS * NUM_EMBEDDINGS), jnp.float32),
        ],
    )(z, codebooks, acb)
    return zq, idx, dist.reshape(bsz, NUM_HEADS, NUM_EMBEDDINGS)


def kernel(z, codebooks):
    return _pq(z, codebooks)


# speedup vs baseline: 4.6300x; 1.3358x over previous
import functools

import jax
import jax.numpy as jnp
from jax.experimental import pallas as pl

NUM_HEADS = 8
NUM_EMBEDDINGS = 512
EMBEDDING_DIM = 256
HEAD_DIM = EMBEDDING_DIM // NUM_HEADS


def _pq_kernel(z_ref, cb_ref, cbt_ref, zq_ref, idx_ref, dist_ref):
    z_blk = z_ref[...]
    zhs = [z_blk[:, h * HEAD_DIM:(h + 1) * HEAD_DIM] for h in range(NUM_HEADS)]
    dots = [jnp.dot(zhs[h], cbt_ref[h], preferred_element_type=jnp.float32)
            for h in range(NUM_HEADS)]
    dists = []
    for h in range(NUM_HEADS):
        zh = zhs[h]
        ch = cb_ref[h]
        z_sq = jnp.sum(zh * zh, axis=-1, keepdims=True)
        c_sq = jnp.sum(ch * ch, axis=-1)[None, :]
        dist = z_sq + c_sq - 2.0 * dots[h]
        dist_ref[:, h * NUM_EMBEDDINGS:(h + 1) * NUM_EMBEDDINGS] = dist
        dists.append(dist)
    idxs = [jnp.argmin(dists[h], axis=-1).astype(jnp.int32) for h in range(NUM_HEADS)]
    idx_ref[...] = jnp.concatenate([i[:, None] for i in idxs], axis=1)
    zq_cols = []
    for h in range(NUM_HEADS):
        onehot = (jax.lax.broadcasted_iota(jnp.int32, dists[h].shape, 1)
                  == idxs[h][:, None]).astype(jnp.float32)
        zq_h = jnp.dot(onehot, cb_ref[h], preferred_element_type=jnp.float32)
        zq_cols.append(zhs[h] + (zq_h - zhs[h]))
    zq_ref[...] = jnp.concatenate(zq_cols, axis=1)


@functools.partial(jax.jit, static_argnames=("block_b",))
def _pq(z, codebooks, block_b=512):
    bsz, dim = z.shape
    cbt = jnp.transpose(codebooks, (0, 2, 1))
    grid = (bsz // block_b,)
    zq, idx, dist = pl.pallas_call(
        _pq_kernel,
        grid=grid,
        in_specs=[
            pl.BlockSpec((block_b, dim), lambda i: (i, 0)),
            pl.BlockSpec((NUM_HEADS, NUM_EMBEDDINGS, HEAD_DIM), lambda i: (0, 0, 0)),
            pl.BlockSpec((NUM_HEADS, HEAD_DIM, NUM_EMBEDDINGS), lambda i: (0, 0, 0)),
        ],
        out_specs=[
            pl.BlockSpec((block_b, dim), lambda i: (i, 0)),
            pl.BlockSpec((block_b, NUM_HEADS), lambda i: (i, 0)),
            pl.BlockSpec((block_b, NUM_HEADS * NUM_EMBEDDINGS), lambda i: (i, 0)),
        ],
        out_shape=[
            jax.ShapeDtypeStruct((bsz, dim), jnp.float32),
            jax.ShapeDtypeStruct((bsz, NUM_HEADS), jnp.int32),
            jax.ShapeDtypeStruct((bsz, NUM_HEADS * NUM_EMBEDDINGS), jnp.float32),
        ],
    )(z, codebooks, cbt)
    return zq, idx, dist.reshape(bsz, NUM_HEADS, NUM_EMBEDDINGS)


def kernel(z, codebooks):
    return _pq(z, codebooks)
